# baseline (speedup 1.0000x reference)
import jax
import jax.numpy as jnp
from jax.experimental import pallas as pl
from jax.experimental.pallas import tpu as pltpu

_BF16 = jnp.bfloat16
_F32 = jnp.float32


def _rvq_kernel(z_ref, ebf_ref, limbs_ref, nrm_ref, out_ref):
    n_books, C, D = ebf_ref.shape
    n_lanes = z_ref.shape[0]
    iota_c = jax.lax.broadcasted_iota(jnp.int32, (C, 1), 0)
    xs = [z_ref[j] for j in range(n_lanes)]
    rs = list(xs)
    for i in range(n_books):
        for j in range(n_lanes):
            scores = jax.lax.dot_general(
                ebf_ref[i], rs[j].astype(_BF16),
                (((1,), (0,)), ((), ())),
                preferred_element_type=_F32)
            scores = scores - nrm_ref[i]
            idx = jnp.argmax(scores, axis=0)
            oh = (iota_c == idx[None, :]).astype(_BF16)
            qcat = jax.lax.dot_general(
                limbs_ref[i], oh, (((1,), (0,)), ((), ())),
                preferred_element_type=_F32)
            q = (qcat[:D] + qcat[D:2 * D]) + qcat[2 * D:]
            rs[j] = rs[j] - q
    for j in range(n_lanes):
        out_ref[j] = xs[j] - rs[j]


def kernel(z, books):
    B, D, T = z.shape
    n_books, C, Dk = books.shape

    ebf = books.astype(_BF16)
    hi = ebf
    r1 = books - hi.astype(_F32)
    mid = r1.astype(_BF16)
    lo = (r1 - mid.astype(_F32)).astype(_BF16)
    limbs_t = jnp.concatenate(
        [jnp.transpose(hi, (0, 2, 1)),
         jnp.transpose(mid, (0, 2, 1)),
         jnp.transpose(lo, (0, 2, 1))], axis=1)
    nrm = 0.5 * jnp.sum(books * books, axis=2)[..., None]

    ilp = 2
    whole = lambda shape: pl.BlockSpec(shape, lambda b: (0,) * len(shape))
    return pl.pallas_call(
        _rvq_kernel,
        grid=(B // ilp,),
        in_specs=[
            pl.BlockSpec((ilp, D, T), lambda b: (b, 0, 0)),
            whole((n_books, C, Dk)),
            whole((n_books, 3 * Dk, C)),
            whole((n_books, C, 1)),
        ],
        out_specs=pl.BlockSpec((ilp, D, T), lambda b: (b, 0, 0)),
        out_shape=jax.ShapeDtypeStruct((B, D, T), z.dtype),
        compiler_params=pltpu.CompilerParams(
            dimension_semantics=("parallel",),
        ),
    )(z, ebf, limbs_t, nrm)


# speedup vs baseline: 1.1729x; 1.1729x over previous
import jax
import jax.numpy as jnp
from jax.experimental import pallas as pl
from jax.experimental.pallas import tpu as pltpu

_BF16 = jnp.bfloat16
_F32 = jnp.float32


def _rvq_kernel(z_ref, fb_ref, out_ref, ebf_s, limbs_s, nrm_s):
    n_books, C, D = fb_ref.shape
    n_lanes = z_ref.shape[0]

    @pl.when(pl.program_id(0) == 0)
    def _prep():
        for i in range(n_books):
            emb = fb_ref[i]
            e_hi = emb.astype(_BF16)
            r1 = emb - e_hi.astype(_F32)
            e_mid = r1.astype(_BF16)
            e_lo = (r1 - e_mid.astype(_F32)).astype(_BF16)
            ebf_s[i] = e_hi
            limbs_s[i, :D] = e_hi.T
            limbs_s[i, D:2 * D] = e_mid.T
            limbs_s[i, 2 * D:] = e_lo.T
            nrm_s[i] = 0.5 * jnp.sum(emb * emb, axis=1, keepdims=True)

    iota_c = jax.lax.broadcasted_iota(jnp.int32, (C, 1), 0)
    xs = [z_ref[j] for j in range(n_lanes)]
    rs = list(xs)
    for i in range(n_books):
        for j in range(n_lanes):
            scores = jax.lax.dot_general(
                ebf_s[i], rs[j].astype(_BF16),
                (((1,), (0,)), ((), ())),
                preferred_element_type=_F32)
            scores = scores - nrm_s[i]
            idx = jnp.argmax(scores, axis=0)
            oh = (iota_c == idx[None, :]).astype(_BF16)
            qcat = jax.lax.dot_general(
                limbs_s[i], oh, (((1,), (0,)), ((), ())),
                preferred_element_type=_F32)
            q = (qcat[:D] + qcat[D:2 * D]) + qcat[2 * D:]
            rs[j] = rs[j] - q
    for j in range(n_lanes):
        out_ref[j] = xs[j] - rs[j]


def kernel(z, books):
    B, D, T = z.shape
    n_books, C, Dk = books.shape
    ilp = 2
    whole = lambda shape: pl.BlockSpec(shape, lambda b: (0,) * len(shape))
    return pl.pallas_call(
        _rvq_kernel,
        grid=(B // ilp,),
        in_specs=[
            pl.BlockSpec((ilp, D, T), lambda b: (b, 0, 0)),
            whole((n_books, C, Dk)),
        ],
        out_specs=pl.BlockSpec((ilp, D, T), lambda b: (b, 0, 0)),
        out_shape=jax.ShapeDtypeStruct((B, D, T), z.dtype),
        scratch_shapes=[
            pltpu.VMEM((n_books, C, Dk), _BF16),
            pltpu.VMEM((n_books, 3 * Dk, C), _BF16),
            pltpu.VMEM((n_books, C, 1), _F32),
        ],
        compiler_params=pltpu.CompilerParams(
            dimension_semantics=("arbitrary",),
        ),
    )(z, books)


# speedup vs baseline: 1.3257x; 1.1303x over previous
import jax
import jax.numpy as jnp
from jax.experimental import pallas as pl
from jax.experimental.pallas import tpu as pltpu

_BF16 = jnp.bfloat16
_F32 = jnp.float32


def _rvq_kernel(z_ref, fb_ref, out_ref, ebf_s, limbs_s, nrm_s):
    n_books, C, D = fb_ref.shape
    n_lanes = z_ref.shape[0]

    @pl.when(pl.program_id(0) == 0)
    def _prep():
        for i in range(n_books):
            emb = fb_ref[i]
            e_hi = emb.astype(_BF16)
            r1 = emb - e_hi.astype(_F32)
            e_mid = r1.astype(_BF16)
            e_lo = (r1 - e_mid.astype(_F32)).astype(_BF16)
            ebf_s[i] = e_hi
            limbs_s[i, :D] = e_hi.T
            limbs_s[i, D:2 * D] = e_mid.T
            limbs_s[i, 2 * D:] = e_lo.T
            nrm_s[i] = 0.5 * jnp.sum(emb * emb, axis=1, keepdims=True)

    iota_c = jax.lax.broadcasted_iota(jnp.int32, (C, 1), 0)
    T = z_ref.shape[2]
    x = jnp.concatenate([z_ref[j] for j in range(n_lanes)], axis=1)
    r = x
    for i in range(n_books):
        scores = jax.lax.dot_general(
            ebf_s[i], r.astype(_BF16),
            (((1,), (0,)), ((), ())),
            preferred_element_type=_F32)
        scores = scores - nrm_s[i]
        idx = jnp.argmax(scores, axis=0)
        oh = (iota_c == idx[None, :]).astype(_BF16)
        qcat = jax.lax.dot_general(
            limbs_s[i], oh, (((1,), (0,)), ((), ())),
            preferred_element_type=_F32)
        q = (qcat[:D] + qcat[D:2 * D]) + qcat[2 * D:]
        r = r - q
    out = x - r
    for j in range(n_lanes):
        out_ref[j] = out[:, j * T:(j + 1) * T]


def kernel(z, books):
    B, D, T = z.shape
    n_books, C, Dk = books.shape
    ilp = 2
    whole = lambda shape: pl.BlockSpec(shape, lambda b: (0,) * len(shape))
    return pl.pallas_call(
        _rvq_kernel,
        grid=(B // ilp,),
        in_specs=[
            pl.BlockSpec((ilp, D, T), lambda b: (b, 0, 0)),
            whole((n_books, C, Dk)),
        ],
        out_specs=pl.BlockSpec((ilp, D, T), lambda b: (b, 0, 0)),
        out_shape=jax.ShapeDtypeStruct((B, D, T), z.dtype),
        scratch_shapes=[
            pltpu.VMEM((n_books, C, Dk), _BF16),
            pltpu.VMEM((n_books, 3 * Dk, C), _BF16),
            pltpu.VMEM((n_books, C, 1), _F32),
        ],
        compiler_params=pltpu.CompilerParams(
            dimension_semantics=("arbitrary",),
        ),
    )(z, books)


# speedup vs baseline: 1.4403x; 1.0864x over previous
import jax
import jax.numpy as jnp
from jax.experimental import pallas as pl
from jax.experimental.pallas import tpu as pltpu

_BF16 = jnp.bfloat16
_F32 = jnp.float32


def _rvq_kernel(z_ref, fb_ref, out_ref, ebf_s, limbs_s, nrm_s):
    n_books, C, D = fb_ref.shape
    n_lanes = z_ref.shape[0]

    @pl.when(pl.program_id(0) == 0)
    def _prep():
        for i in range(n_books):
            emb = fb_ref[i]
            e_hi = emb.astype(_BF16)
            r1 = emb - e_hi.astype(_F32)
            e_mid = r1.astype(_BF16)
            e_lo = (r1 - e_mid.astype(_F32)).astype(_BF16)
            ebf_s[i] = e_hi
            limbs_s[i, :D] = e_hi.T
            limbs_s[i, D:2 * D] = e_mid.T
            limbs_s[i, 2 * D:] = e_lo.T
            nrm_s[i] = 0.5 * jnp.sum(emb * emb, axis=1, keepdims=True)

    iota_c = jax.lax.broadcasted_iota(jnp.int32, (C, 1), 0)
    T = z_ref.shape[2]
    x = jnp.concatenate([z_ref[j] for j in range(n_lanes)], axis=1)
    r = x
    for i in range(n_books):
        scores = jax.lax.dot_general(
            ebf_s[i], r.astype(_BF16),
            (((1,), (0,)), ((), ())),
            preferred_element_type=_F32)
        scores = scores - nrm_s[i]
        idx = jnp.argmax(scores, axis=0)
        oh = (iota_c == idx[None, :]).astype(_BF16)
        qcat = jax.lax.dot_general(
            limbs_s[i], oh, (((1,), (0,)), ((), ())),
            preferred_element_type=_F32)
        q = (qcat[:D] + qcat[D:2 * D]) + qcat[2 * D:]
        r = r - q
    out = x - r
    for j in range(n_lanes):
        out_ref[j] = out[:, j * T:(j + 1) * T]


def kernel(z, books):
    B, D, T = z.shape
    n_books, C, Dk = books.shape
    ilp = 4
    whole = lambda shape: pl.BlockSpec(shape, lambda b: (0,) * len(shape))
    return pl.pallas_call(
        _rvq_kernel,
        grid=(B // ilp,),
        in_specs=[
            pl.BlockSpec((ilp, D, T), lambda b: (b, 0, 0)),
            whole((n_books, C, Dk)),
        ],
        out_specs=pl.BlockSpec((ilp, D, T), lambda b: (b, 0, 0)),
        out_shape=jax.ShapeDtypeStruct((B, D, T), z.dtype),
        scratch_shapes=[
            pltpu.VMEM((n_books, C, Dk), _BF16),
            pltpu.VMEM((n_books, 3 * Dk, C), _BF16),
            pltpu.VMEM((n_books, C, 1), _F32),
        ],
        compiler_params=pltpu.CompilerParams(
            dimension_semantics=("arbitrary",),
        ),
    )(z, books)
